# baseline (speedup 1.0000x reference)
import functools
import jax
import jax.numpy as jnp
from jax import lax
from jax.experimental import pallas as pl
from jax.experimental.pallas import tpu as pltpu
from jax.experimental.pallas import tpu_sc as plsc

_V = 1000000
_D = 32
_L = 16


def _iota():
    return lax.iota(jnp.int32, _L)


def _make_transpose():
    info = plsc.get_sparse_core_info()
    NC, NS = info.num_cores, info.num_subcores
    NW = NC * NS
    SB = 512
    J = _V // SB
    ROWS = _V * _D // 128
    RB = SB * _D // 128
    mesh = plsc.VectorSubcoreMesh(core_axis_name="c", subcore_axis_name="s")

    @functools.partial(
        pl.kernel,
        mesh=mesh,
        out_type=jax.ShapeDtypeStruct((ROWS, 128), jnp.float32),
        compiler_params=pltpu.CompilerParams(needs_layout_passes=False),
        scratch_types=[
            pltpu.VMEM((_D, SB), jnp.float32),
            pltpu.VMEM((_D, SB), jnp.float32),
            pltpu.VMEM((RB, 128), jnp.float32),
            pltpu.VMEM((RB, 128), jnp.float32),
            pltpu.SemaphoreType.DMA,
            pltpu.SemaphoreType.DMA,
            pltpu.SemaphoreType.DMA,
            pltpu.SemaphoreType.DMA,
        ],
    )
    def k(wt_hbm, tail_hbm, wrow_hbm, a0, a1, t0, t1, g0, g1, o0, o1):
        wid = lax.axis_index("s") * NC + lax.axis_index("c")
        start = wid * 61 + jnp.minimum(wid, 1)
        abufs, tbufs, gsems, osems = (a0, a1), (t0, t1), (g0, g1), (o0, o1)
        iot = _iota()
        rows_bg = [4 * bg + (iot >> 2) for bg in range(8)]
        colk = iot & 3

        def transpose_block(abuf, tbuf):
            @plsc.parallel_loop(0, RB, unroll=8)
            def _(a):
                av = colk + 4 * a
                for bg in range(8):
                    v = plsc.load_gather(abuf, [rows_bg[bg], av])
                    tbuf[a, pl.ds(bg * _L, _L)] = v

        @pl.loop(0, 31)
        def grp(i):
            js = []
            for b in range(2):
                j = jnp.minimum(start + 2 * i + b, J - 1)
                js.append(j)

                @pl.when(i > 0)
                def _():
                    pltpu.make_async_copy(
                        tbufs[b], wrow_hbm.at[pl.ds(j * RB, RB), :],
                        osems[b]).wait()
                pltpu.async_copy(wt_hbm.at[:, pl.ds(j * SB, SB)],
                                 abufs[b], gsems[b])
            for b in range(2):
                pltpu.make_async_copy(
                    wt_hbm.at[:, pl.ds(js[b] * SB, SB)], abufs[b],
                    gsems[b]).wait()
                transpose_block(abufs[b], tbufs[b])
                pltpu.async_copy(tbufs[b],
                                 wrow_hbm.at[pl.ds(js[b] * RB, RB), :],
                                 osems[b])

        for b in range(2):
            pltpu.make_async_copy(
                tbufs[b], wrow_hbm.at[pl.ds(0, RB), :], osems[b]).wait()

        @pl.when(wid == NW - 1)
        def _tail():
            pltpu.sync_copy(tail_hbm, t0.at[pl.ds(0, 16), :])
            pltpu.sync_copy(t0.at[pl.ds(0, 16), :],
                            wrow_hbm.at[pl.ds(J * RB, 16), :])

    return k


def _make_gather(C, B):
    info = plsc.get_sparse_core_info()
    NC, NS = info.num_cores, info.num_subcores
    NW = NC * NS
    BW = B // NW
    CH = 256
    NB = BW // CH
    mesh = plsc.VectorSubcoreMesh(core_axis_name="c", subcore_axis_name="s")

    @functools.partial(
        pl.kernel,
        mesh=mesh,
        out_type=jax.ShapeDtypeStruct((C, _D, B), jnp.float32),
        compiler_params=pltpu.CompilerParams(needs_layout_passes=False),
        scratch_types=[
            pltpu.VMEM((BW,), jnp.int32),
            pltpu.VMEM((BW,), jnp.int32),
            pltpu.VMEM((CH, 128), jnp.float32),
            pltpu.VMEM((CH, 128), jnp.float32),
            pltpu.VMEM((_D, CH), jnp.float32),
            pltpu.VMEM((_D, CH), jnp.float32),
            pltpu.SemaphoreType.DMA,
            pltpu.SemaphoreType.DMA,
            pltpu.SemaphoreType.DMA,
            pltpu.SemaphoreType.DMA,
        ],
    )
    def k(xt_hbm, wrow_hbm, out_hbm, ibuf, pbuf,
          gb0, gb1, t0, t1, g0, g1, o0, o1):
        wid = lax.axis_index("s") * NC + lax.axis_index("c")
        base = wid * BW
        gbufs, tbufs = (gb0, gb1), (t0, t1)
        gsems, osems = (g0, g1), (o0, o1)
        iot = _iota()

        def extract_chunk(b, gbuf, tbuf):
            @plsc.parallel_loop(0, CH // _L, unroll=4)
            def _(bg):
                rvec = ibuf[pl.ds(b * CH + bg * _L, _L)]
                cbase = rvec & 3
                rows = iot + bg * _L
                for d in range(_D):
                    v = plsc.load_gather(gbuf, [rows, cbase + 4 * d])
                    tbuf[d, pl.ds(bg * _L, _L)] = v

        @pl.loop(0, C)
        def col(c):
            for b in range(NB):
                @pl.when(c > 0)
                def _():
                    pltpu.make_async_copy(
                        tbufs[b],
                        out_hbm.at[c, :, pl.ds(base + b * CH, CH)],
                        osems[b]).wait()
            pltpu.sync_copy(xt_hbm.at[c, pl.ds(base, BW)], ibuf)

            @plsc.parallel_loop(0, BW // _L, unroll=2)
            def _(g):
                pbuf[pl.ds(g * _L, _L)] = ibuf[pl.ds(g * _L, _L)] >> 2

            for b in range(NB):
                pltpu.async_copy(
                    wrow_hbm.at[pbuf.at[pl.ds(b * CH, CH)]], gbufs[b],
                    gsems[b])
            for b in range(NB):
                pltpu.make_async_copy(
                    wrow_hbm.at[pbuf.at[pl.ds(b * CH, CH)]], gbufs[b],
                    gsems[b]).wait()
                extract_chunk(b, gbufs[b], tbufs[b])
                pltpu.async_copy(
                    tbufs[b], out_hbm.at[c, :, pl.ds(base + b * CH, CH)],
                    osems[b])

        for b in range(NB):
            pltpu.make_async_copy(
                tbufs[b], out_hbm.at[C - 1, :, pl.ds(base + b * CH, CH)],
                osems[b]).wait()

    return k


def kernel(x, weight):
    B, C = x.shape
    xt = x.T.astype(jnp.int32)
    wt = weight.T
    tail = weight[(_V // 128) * 128:].reshape(16, 4, _D)
    tail = jnp.transpose(tail, (0, 2, 1)).reshape(16, 128)
    wrow = _make_transpose()(wt, tail)
    out5 = _make_gather(C, B)(xt, wrow)
    return jnp.transpose(out5, (2, 0, 1))


# speedup vs baseline: 1.0352x; 1.0352x over previous
import functools
import jax
import jax.numpy as jnp
from jax import lax
from jax.experimental import pallas as pl
from jax.experimental.pallas import tpu as pltpu
from jax.experimental.pallas import tpu_sc as plsc

_V = 1000000
_D = 32
_L = 16


def _iota():
    return lax.iota(jnp.int32, _L)


def _make_transpose():
    info = plsc.get_sparse_core_info()
    NC, NS = info.num_cores, info.num_subcores
    NW = NC * NS
    SB = 512
    J = _V // SB
    ROWS = _V * _D // 128
    RB = SB * _D // 128
    mesh = plsc.VectorSubcoreMesh(core_axis_name="c", subcore_axis_name="s")

    @functools.partial(
        pl.kernel,
        mesh=mesh,
        out_type=jax.ShapeDtypeStruct((ROWS, 128), jnp.float32),
        compiler_params=pltpu.CompilerParams(needs_layout_passes=False),
        scratch_types=[
            pltpu.VMEM((_D, SB), jnp.float32),
            pltpu.VMEM((_D, SB), jnp.float32),
            pltpu.VMEM((RB, 128), jnp.float32),
            pltpu.VMEM((RB, 128), jnp.float32),
            pltpu.SemaphoreType.DMA,
            pltpu.SemaphoreType.DMA,
            pltpu.SemaphoreType.DMA,
            pltpu.SemaphoreType.DMA,
        ],
    )
    def k(wt_hbm, tail_hbm, wrow_hbm, a0, a1, t0, t1, g0, g1, o0, o1):
        wid = lax.axis_index("s") * NC + lax.axis_index("c")
        start = wid * 61 + jnp.minimum(wid, 1)
        abufs, tbufs, gsems, osems = (a0, a1), (t0, t1), (g0, g1), (o0, o1)
        iot = _iota()
        rows_bg = [4 * bg + (iot >> 2) for bg in range(8)]
        colk = iot & 3

        def transpose_block(abuf, tbuf):
            @plsc.parallel_loop(0, RB, unroll=8)
            def _(a):
                av = colk + 4 * a
                for bg in range(8):
                    v = plsc.load_gather(abuf, [rows_bg[bg], av])
                    tbuf[a, pl.ds(bg * _L, _L)] = v

        @pl.loop(0, 31)
        def grp(i):
            js = []
            for b in range(2):
                j = jnp.minimum(start + 2 * i + b, J - 1)
                js.append(j)

                @pl.when(i > 0)
                def _():
                    pltpu.make_async_copy(
                        tbufs[b], wrow_hbm.at[pl.ds(j * RB, RB), :],
                        osems[b]).wait()
                pltpu.async_copy(wt_hbm.at[:, pl.ds(j * SB, SB)],
                                 abufs[b], gsems[b])
            for b in range(2):
                pltpu.make_async_copy(
                    wt_hbm.at[:, pl.ds(js[b] * SB, SB)], abufs[b],
                    gsems[b]).wait()
                transpose_block(abufs[b], tbufs[b])
                pltpu.async_copy(tbufs[b],
                                 wrow_hbm.at[pl.ds(js[b] * RB, RB), :],
                                 osems[b])

        for b in range(2):
            pltpu.make_async_copy(
                tbufs[b], wrow_hbm.at[pl.ds(0, RB), :], osems[b]).wait()

        @pl.when(wid == NW - 1)
        def _tail():
            pltpu.sync_copy(tail_hbm, t0.at[pl.ds(0, 16), :])
            pltpu.sync_copy(t0.at[pl.ds(0, 16), :],
                            wrow_hbm.at[pl.ds(J * RB, 16), :])

    return k


def _make_gather(C, B):
    info = plsc.get_sparse_core_info()
    NC, NS = info.num_cores, info.num_subcores
    NW = NC * NS
    BW = B // NW
    CH = 128
    NB = BW // CH
    mesh = plsc.VectorSubcoreMesh(core_axis_name="c", subcore_axis_name="s")

    @functools.partial(
        pl.kernel,
        mesh=mesh,
        out_type=jax.ShapeDtypeStruct((C, _D, B), jnp.float32),
        compiler_params=pltpu.CompilerParams(needs_layout_passes=False),
        scratch_types=[
            pltpu.VMEM((BW,), jnp.int32),
            pltpu.VMEM((BW,), jnp.int32),
            pltpu.VMEM((CH, 128), jnp.float32),
            pltpu.VMEM((CH, 128), jnp.float32),
            pltpu.VMEM((CH, 128), jnp.float32),
            pltpu.VMEM((CH, 128), jnp.float32),
            pltpu.VMEM((_D, CH), jnp.float32),
            pltpu.VMEM((_D, CH), jnp.float32),
            pltpu.VMEM((_D, CH), jnp.float32),
            pltpu.VMEM((_D, CH), jnp.float32),
            pltpu.SemaphoreType.DMA,
            pltpu.SemaphoreType.DMA,
            pltpu.SemaphoreType.DMA,
            pltpu.SemaphoreType.DMA,
            pltpu.SemaphoreType.DMA,
            pltpu.SemaphoreType.DMA,
            pltpu.SemaphoreType.DMA,
            pltpu.SemaphoreType.DMA,
        ],
    )
    def k(xt_hbm, wrow_hbm, out_hbm, ibuf, pbuf,
          gb0, gb1, gb2, gb3, t0, t1, t2, t3,
          g0, g1, g2, g3, o0, o1, o2, o3):
        wid = lax.axis_index("s") * NC + lax.axis_index("c")
        base = wid * BW
        gbufs, tbufs = (gb0, gb1, gb2, gb3), (t0, t1, t2, t3)
        gsems, osems = (g0, g1, g2, g3), (o0, o1, o2, o3)
        iot = _iota()

        def extract_chunk(b, gbuf, tbuf):
            @plsc.parallel_loop(0, CH // _L, unroll=4)
            def _(bg):
                rvec = ibuf[pl.ds(b * CH + bg * _L, _L)]
                cbase = rvec & 3
                rows = iot + bg * _L
                for d in range(_D):
                    v = plsc.load_gather(gbuf, [rows, cbase + 4 * d])
                    tbuf[d, pl.ds(bg * _L, _L)] = v

        @pl.loop(0, C)
        def col(c):
            for b in range(NB):
                @pl.when(c > 0)
                def _():
                    pltpu.make_async_copy(
                        tbufs[b],
                        out_hbm.at[c, :, pl.ds(base + b * CH, CH)],
                        osems[b]).wait()
            pltpu.sync_copy(xt_hbm.at[c, pl.ds(base, BW)], ibuf)

            @plsc.parallel_loop(0, BW // _L, unroll=2)
            def _(g):
                pbuf[pl.ds(g * _L, _L)] = ibuf[pl.ds(g * _L, _L)] >> 2

            for b in range(NB):
                pltpu.async_copy(
                    wrow_hbm.at[pbuf.at[pl.ds(b * CH, CH)]], gbufs[b],
                    gsems[b])
            for b in range(NB):
                pltpu.make_async_copy(
                    wrow_hbm.at[pbuf.at[pl.ds(b * CH, CH)]], gbufs[b],
                    gsems[b]).wait()
                extract_chunk(b, gbufs[b], tbufs[b])
                pltpu.async_copy(
                    tbufs[b], out_hbm.at[c, :, pl.ds(base + b * CH, CH)],
                    osems[b])

        for b in range(NB):
            pltpu.make_async_copy(
                tbufs[b], out_hbm.at[C - 1, :, pl.ds(base + b * CH, CH)],
                osems[b]).wait()

    return k


def kernel(x, weight):
    B, C = x.shape
    xt = x.T.astype(jnp.int32)
    wt = weight.T
    tail = weight[(_V // 128) * 128:].reshape(16, 4, _D)
    tail = jnp.transpose(tail, (0, 2, 1)).reshape(16, 128)
    wrow = _make_transpose()(wt, tail)
    out5 = _make_gather(C, B)(xt, wrow)
    return jnp.transpose(out5, (2, 0, 1))
